# baseline (device time: 214861 ns/iter reference)
import jax
import jax.numpy as jnp
from jax import lax
from jax.experimental import pallas as pl
from jax.experimental.pallas import tpu as pltpu

N_Y = 4
B, S, H, Dh, Dr = 2, 256, 16, 64, 32
D = 1024
BS = B * S
SCALE = (Dh + Dr) ** -0.5


def kernel(x, Wdkv, Wuk, Wuv, Wq, Wqr, Wkr, Wo):
    x2 = x.reshape(BS, D)

    def body(x_ref, wdkv_ref, wuk_ref, wuv_ref, wq_ref, wqr_ref, wkr_ref,
             wo_ref, out_ref, kv_ref, o_ref, send_sems, recv_sems):
        my_x = lax.axis_index("x")
        my_y = lax.axis_index("y")
        my_z = lax.axis_index("z")

        xv = x_ref[...]
        c = jnp.dot(xv, wdkv_ref[...], preferred_element_type=jnp.float32)
        k_part = jnp.dot(c, wuk_ref[...], preferred_element_type=jnp.float32)
        v_part = jnp.dot(c, wuv_ref[...], preferred_element_type=jnp.float32)
        kv_ref[my_y] = jnp.concatenate([k_part, v_part], axis=-1)

        barrier = pltpu.get_barrier_semaphore()
        for p in range(N_Y):
            @pl.when(my_y != p)
            def _():
                pl.semaphore_signal(
                    barrier, inc=1,
                    device_id=(my_x, p, my_z),
                    device_id_type=pl.DeviceIdType.MESH,
                )
        pl.semaphore_wait(barrier, N_Y - 1)

        for p in range(N_Y):
            @pl.when(my_y != p)
            def _():
                rdma = pltpu.make_async_remote_copy(
                    src_ref=kv_ref.at[my_y],
                    dst_ref=kv_ref.at[my_y],
                    send_sem=send_sems.at[p],
                    recv_sem=recv_sems.at[my_y],
                    device_id=(my_x, p, my_z),
                    device_id_type=pl.DeviceIdType.MESH,
                )
                rdma.start()

        q_all = jnp.dot(xv, wq_ref[...], preferred_element_type=jnp.float32)
        qr_all = jnp.dot(xv, wqr_ref[...], preferred_element_type=jnp.float32)
        kr_all = jnp.dot(xv, wkr_ref[...], preferred_element_type=jnp.float32)

        for p in range(N_Y):
            @pl.when(my_y != p)
            def _():
                recv = pltpu.make_async_remote_copy(
                    src_ref=kv_ref.at[p],
                    dst_ref=kv_ref.at[p],
                    send_sem=send_sems.at[p],
                    recv_sem=recv_sems.at[p],
                    device_id=(my_x, p, my_z),
                    device_id_type=pl.DeviceIdType.MESH,
                )
                recv.wait_recv()

        kv = kv_ref[0] + kv_ref[1] + kv_ref[2] + kv_ref[3]
        k_all = kv[:, :D]
        v_all = kv[:, D:]

        for b in range(B):
            kr_b = kr_all[b * S:(b + 1) * S, :]
            for h in range(H):
                q = q_all[b * S:(b + 1) * S, h * Dh:(h + 1) * Dh]
                qr = qr_all[b * S:(b + 1) * S, h * Dr:(h + 1) * Dr]
                k = k_all[b * S:(b + 1) * S, h * Dh:(h + 1) * Dh]
                v = v_all[b * S:(b + 1) * S, h * Dh:(h + 1) * Dh]
                scores = (
                    lax.dot_general(q, k, (((1,), (1,)), ((), ())),
                                    preferred_element_type=jnp.float32)
                    + lax.dot_general(qr, kr_b, (((1,), (1,)), ((), ())),
                                      preferred_element_type=jnp.float32)
                ) * SCALE
                m = jnp.max(scores, axis=-1, keepdims=True)
                e = jnp.exp(scores - m)
                pattn = e / jnp.sum(e, axis=-1, keepdims=True)
                o = jnp.dot(pattn, v, preferred_element_type=jnp.float32)
                o_ref[b * S:(b + 1) * S, h * Dh:(h + 1) * Dh] = o

        out_ref[...] = jnp.dot(o_ref[...], wo_ref[...],
                               preferred_element_type=jnp.float32)

        for p in range(N_Y):
            @pl.when(my_y != p)
            def _():
                snd = pltpu.make_async_remote_copy(
                    src_ref=kv_ref.at[my_y],
                    dst_ref=kv_ref.at[my_y],
                    send_sem=send_sems.at[p],
                    recv_sem=recv_sems.at[p],
                    device_id=(my_x, p, my_z),
                    device_id_type=pl.DeviceIdType.MESH,
                )
                snd.wait_send()

    out = pl.pallas_call(
        body,
        out_shape=jax.ShapeDtypeStruct((BS, D), jnp.float32),
        in_specs=[pl.BlockSpec(memory_space=pltpu.VMEM)] * 8,
        out_specs=pl.BlockSpec(memory_space=pltpu.VMEM),
        scratch_shapes=[
            pltpu.VMEM((N_Y, BS, 2 * D), jnp.float32),
            pltpu.VMEM((BS, D), jnp.float32),
            pltpu.SemaphoreType.DMA((N_Y,)),
            pltpu.SemaphoreType.DMA((N_Y,)),
        ],
        compiler_params=pltpu.CompilerParams(collective_id=0),
    )(x2, Wdkv, Wuk, Wuv, Wq, Wqr, Wkr, Wo)
    return out.reshape(B, S, D)


# device time: 61991 ns/iter; 3.4660x vs baseline; 3.4660x over previous
import jax
import jax.numpy as jnp
from jax import lax
from jax.experimental import pallas as pl
from jax.experimental.pallas import tpu as pltpu

N_Y = 4
B, S, H, Dh, Dr = 2, 256, 16, 64, 32
D = 1024
BS = B * S
HL = H // N_Y
CW = HL * Dh
RW = HL * Dr
DC = 64
SCALE = (Dh + Dr) ** -0.5


def kernel(x, Wdkv, Wuk, Wuv, Wq, Wqr, Wkr, Wo):
    x2 = x.reshape(BS, D)

    def body(x_ref, wdkv_ref, wuk_ref, wuv_ref, wq_ref, wqr_ref, wkr_ref,
             wo_ref, out_ref, cbuf, wkbuf, wvbuf, obuf,
             c_s, c_r, wk_s, wk_r, wv_s, wv_r, o_s, o_r):
        my_x = lax.axis_index("x")
        my_y = lax.axis_index("y")
        my_z = lax.axis_index("z")

        xv = x_ref[...]

        cbuf[my_y] = jnp.dot(xv, wdkv_ref[...],
                             preferred_element_type=jnp.float32)
        wkbuf[my_y] = wuk_ref[:, pl.ds(my_y * CW, CW)]
        wvbuf[my_y] = wuv_ref[:, pl.ds(my_y * CW, CW)]

        barrier = pltpu.get_barrier_semaphore()
        for p in range(N_Y):
            @pl.when(my_y != p)
            def _():
                pl.semaphore_signal(
                    barrier, inc=1,
                    device_id=(my_x, p, my_z),
                    device_id_type=pl.DeviceIdType.MESH,
                )
        pl.semaphore_wait(barrier, N_Y - 1)

        for p in range(N_Y):
            @pl.when(my_y != p)
            def _():
                dev = (my_x, p, my_z)
                pltpu.make_async_remote_copy(
                    src_ref=cbuf.at[my_y], dst_ref=cbuf.at[my_y],
                    send_sem=c_s.at[p], recv_sem=c_r.at[my_y],
                    device_id=dev, device_id_type=pl.DeviceIdType.MESH,
                ).start()
                pltpu.make_async_remote_copy(
                    src_ref=wuk_ref.at[:, pl.ds(p * CW, CW)],
                    dst_ref=wkbuf.at[my_y],
                    send_sem=wk_s.at[p], recv_sem=wk_r.at[my_y],
                    device_id=dev, device_id_type=pl.DeviceIdType.MESH,
                ).start()
                pltpu.make_async_remote_copy(
                    src_ref=wuv_ref.at[:, pl.ds(p * CW, CW)],
                    dst_ref=wvbuf.at[my_y],
                    send_sem=wv_s.at[p], recv_sem=wv_r.at[my_y],
                    device_id=dev, device_id_type=pl.DeviceIdType.MESH,
                ).start()

        q_y = jnp.dot(xv, wq_ref[:, pl.ds(my_y * CW, CW)],
                      preferred_element_type=jnp.float32)
        qr_y = jnp.dot(xv, wqr_ref[:, pl.ds(my_y * RW, RW)],
                       preferred_element_type=jnp.float32)
        kr_all = jnp.dot(xv, wkr_ref[...],
                         preferred_element_type=jnp.float32)

        for p in range(N_Y):
            @pl.when(my_y != p)
            def _():
                dev = (my_x, p, my_z)
                pltpu.make_async_remote_copy(
                    src_ref=cbuf.at[p], dst_ref=cbuf.at[p],
                    send_sem=c_s.at[p], recv_sem=c_r.at[p],
                    device_id=dev, device_id_type=pl.DeviceIdType.MESH,
                ).wait_recv()
                pltpu.make_async_remote_copy(
                    src_ref=wkbuf.at[p], dst_ref=wkbuf.at[p],
                    send_sem=wk_s.at[p], recv_sem=wk_r.at[p],
                    device_id=dev, device_id_type=pl.DeviceIdType.MESH,
                ).wait_recv()
                pltpu.make_async_remote_copy(
                    src_ref=wvbuf.at[p], dst_ref=wvbuf.at[p],
                    send_sem=wv_s.at[p], recv_sem=wv_r.at[p],
                    device_id=dev, device_id_type=pl.DeviceIdType.MESH,
                ).wait_recv()

        c_full = jnp.concatenate([cbuf[p] for p in range(N_Y)], axis=1)
        wk_my = jnp.concatenate([wkbuf[p] for p in range(N_Y)], axis=0)
        wv_my = jnp.concatenate([wvbuf[p] for p in range(N_Y)], axis=0)
        k_y = jnp.dot(c_full, wk_my, preferred_element_type=jnp.float32)
        v_y = jnp.dot(c_full, wv_my, preferred_element_type=jnp.float32)

        for b in range(B):
            rows = pl.ds(b * S, S)
            kr_b = kr_all[b * S:(b + 1) * S, :]
            for h in range(HL):
                q = q_y[b * S:(b + 1) * S, h * Dh:(h + 1) * Dh]
                qr = qr_y[b * S:(b + 1) * S, h * Dr:(h + 1) * Dr]
                k = k_y[b * S:(b + 1) * S, h * Dh:(h + 1) * Dh]
                v = v_y[b * S:(b + 1) * S, h * Dh:(h + 1) * Dh]
                scores = (
                    lax.dot_general(q, k, (((1,), (1,)), ((), ())),
                                    preferred_element_type=jnp.float32)
                    + lax.dot_general(qr, kr_b, (((1,), (1,)), ((), ())),
                                      preferred_element_type=jnp.float32)
                ) * SCALE
                m = jnp.max(scores, axis=-1, keepdims=True)
                e = jnp.exp(scores - m)
                pattn = e / jnp.sum(e, axis=-1, keepdims=True)
                o = jnp.dot(pattn, v, preferred_element_type=jnp.float32)
                obuf[my_y, rows, pl.ds(h * Dh, Dh)] = o

        for p in range(N_Y):
            @pl.when(my_y != p)
            def _():
                pltpu.make_async_remote_copy(
                    src_ref=obuf.at[my_y], dst_ref=obuf.at[my_y],
                    send_sem=o_s.at[p], recv_sem=o_r.at[my_y],
                    device_id=(my_x, p, my_z),
                    device_id_type=pl.DeviceIdType.MESH,
                ).start()

        out_ref[...] = jnp.dot(
            obuf[my_y], wo_ref[pl.ds(my_y * CW, CW), :],
            preferred_element_type=jnp.float32)

        for p in range(N_Y):
            @pl.when(my_y != p)
            def _():
                pltpu.make_async_remote_copy(
                    src_ref=obuf.at[p], dst_ref=obuf.at[p],
                    send_sem=o_s.at[p], recv_sem=o_r.at[p],
                    device_id=(my_x, p, my_z),
                    device_id_type=pl.DeviceIdType.MESH,
                ).wait_recv()
                out_ref[...] = out_ref[...] + jnp.dot(
                    obuf[p], wo_ref[pl.ds(p * CW, CW), :],
                    preferred_element_type=jnp.float32)

        for p in range(N_Y):
            @pl.when(my_y != p)
            def _():
                dev = (my_x, p, my_z)
                for sem, buf in ((c_s, cbuf), (wk_s, wkbuf),
                                 (wv_s, wvbuf), (o_s, obuf)):
                    pltpu.make_async_remote_copy(
                        src_ref=buf.at[my_y], dst_ref=buf.at[my_y],
                        send_sem=sem.at[p], recv_sem=o_r.at[p],
                        device_id=dev, device_id_type=pl.DeviceIdType.MESH,
                    ).wait_send()

    out = pl.pallas_call(
        body,
        out_shape=jax.ShapeDtypeStruct((BS, D), jnp.float32),
        in_specs=[pl.BlockSpec(memory_space=pltpu.VMEM)] * 8,
        out_specs=pl.BlockSpec(memory_space=pltpu.VMEM),
        scratch_shapes=[
            pltpu.VMEM((N_Y, BS, DC), jnp.float32),
            pltpu.VMEM((N_Y, DC, CW), jnp.float32),
            pltpu.VMEM((N_Y, DC, CW), jnp.float32),
            pltpu.VMEM((N_Y, BS, CW), jnp.float32),
            pltpu.SemaphoreType.DMA((N_Y,)),
            pltpu.SemaphoreType.DMA((N_Y,)),
            pltpu.SemaphoreType.DMA((N_Y,)),
            pltpu.SemaphoreType.DMA((N_Y,)),
            pltpu.SemaphoreType.DMA((N_Y,)),
            pltpu.SemaphoreType.DMA((N_Y,)),
            pltpu.SemaphoreType.DMA((N_Y,)),
            pltpu.SemaphoreType.DMA((N_Y,)),
        ],
        compiler_params=pltpu.CompilerParams(collective_id=0),
    )(x2, Wdkv, Wuk, Wuv, Wq, Wqr, Wkr, Wo)
    return out.reshape(B, S, D)


# device time: 45124 ns/iter; 4.7616x vs baseline; 1.3738x over previous
import jax
import jax.numpy as jnp
from jax import lax
from jax.experimental import pallas as pl
from jax.experimental.pallas import tpu as pltpu

N_Y = 4
B, S, H, Dh, Dr = 2, 256, 16, 64, 32
D = 1024
BS = B * S
HL = H // N_Y
CW = HL * Dh
RW = HL * Dr
DC = 64
SCALE = (Dh + Dr) ** -0.5
BF = jnp.bfloat16
F32 = jnp.float32


def kernel(x, Wdkv, Wuk, Wuv, Wq, Wqr, Wkr, Wo):
    x2 = x.reshape(BS, D)

    def body(x_ref, wdkv_ref, wuk_ref, wuv_ref, wq_ref, wqr_ref, wkr_ref,
             wo_ref, out_ref, cbuf, wkbuf, wvbuf, obuf,
             c_s, c_r, wk_s, wk_r, wv_s, wv_r, o_s, o_r):
        my_x = lax.axis_index("x")
        my_y = lax.axis_index("y")
        my_z = lax.axis_index("z")

        xv = x_ref[...].astype(BF)

        cbuf[my_y] = jnp.dot(xv, wdkv_ref[...].astype(BF),
                             preferred_element_type=F32).astype(BF)
        wkbuf[my_y] = wuk_ref[:, pl.ds(my_y * CW, CW)]
        wvbuf[my_y] = wuv_ref[:, pl.ds(my_y * CW, CW)]

        barrier = pltpu.get_barrier_semaphore()
        for p in range(N_Y):
            @pl.when(my_y != p)
            def _():
                pl.semaphore_signal(
                    barrier, inc=1,
                    device_id=(my_x, p, my_z),
                    device_id_type=pl.DeviceIdType.MESH,
                )
        pl.semaphore_wait(barrier, N_Y - 1)

        for p in range(N_Y):
            @pl.when(my_y != p)
            def _():
                dev = (my_x, p, my_z)
                pltpu.make_async_remote_copy(
                    src_ref=cbuf.at[my_y], dst_ref=cbuf.at[my_y],
                    send_sem=c_s.at[p], recv_sem=c_r.at[my_y],
                    device_id=dev, device_id_type=pl.DeviceIdType.MESH,
                ).start()
                pltpu.make_async_remote_copy(
                    src_ref=wuk_ref.at[:, pl.ds(p * CW, CW)],
                    dst_ref=wkbuf.at[my_y],
                    send_sem=wk_s.at[p], recv_sem=wk_r.at[my_y],
                    device_id=dev, device_id_type=pl.DeviceIdType.MESH,
                ).start()
                pltpu.make_async_remote_copy(
                    src_ref=wuv_ref.at[:, pl.ds(p * CW, CW)],
                    dst_ref=wvbuf.at[my_y],
                    send_sem=wv_s.at[p], recv_sem=wv_r.at[my_y],
                    device_id=dev, device_id_type=pl.DeviceIdType.MESH,
                ).start()

        q_y = jnp.dot(xv, wq_ref[:, pl.ds(my_y * CW, CW)].astype(BF),
                      preferred_element_type=F32)
        qr_y = jnp.dot(xv, wqr_ref[:, pl.ds(my_y * RW, RW)].astype(BF),
                       preferred_element_type=F32)
        kr_all = jnp.dot(xv, wkr_ref[...].astype(BF),
                         preferred_element_type=F32)
        q_y = q_y.astype(BF)
        qr_y = qr_y.astype(BF)
        kr_bf = kr_all.astype(BF)

        for p in range(N_Y):
            @pl.when(my_y != p)
            def _():
                dev = (my_x, p, my_z)
                pltpu.make_async_remote_copy(
                    src_ref=cbuf.at[p], dst_ref=cbuf.at[p],
                    send_sem=c_s.at[p], recv_sem=c_r.at[p],
                    device_id=dev, device_id_type=pl.DeviceIdType.MESH,
                ).wait_recv()
                pltpu.make_async_remote_copy(
                    src_ref=wkbuf.at[p], dst_ref=wkbuf.at[p],
                    send_sem=wk_s.at[p], recv_sem=wk_r.at[p],
                    device_id=dev, device_id_type=pl.DeviceIdType.MESH,
                ).wait_recv()
                pltpu.make_async_remote_copy(
                    src_ref=wvbuf.at[p], dst_ref=wvbuf.at[p],
                    send_sem=wv_s.at[p], recv_sem=wv_r.at[p],
                    device_id=dev, device_id_type=pl.DeviceIdType.MESH,
                ).wait_recv()

        c_full = jnp.concatenate([cbuf[p] for p in range(N_Y)], axis=1)
        wk_my = jnp.concatenate(
            [wkbuf[p] for p in range(N_Y)], axis=0).astype(BF)
        wv_my = jnp.concatenate(
            [wvbuf[p] for p in range(N_Y)], axis=0).astype(BF)
        k_y = jnp.dot(c_full, wk_my, preferred_element_type=F32).astype(BF)
        v_y = jnp.dot(c_full, wv_my, preferred_element_type=F32).astype(BF)

        for b in range(B):
            kr_b = kr_bf[b * S:(b + 1) * S, :]
            for h in range(HL):
                q = q_y[b * S:(b + 1) * S, h * Dh:(h + 1) * Dh]
                qr = qr_y[b * S:(b + 1) * S, h * Dr:(h + 1) * Dr]
                k = k_y[b * S:(b + 1) * S, h * Dh:(h + 1) * Dh]
                v = v_y[b * S:(b + 1) * S, h * Dh:(h + 1) * Dh]
                scores = (
                    lax.dot_general(q, k, (((1,), (1,)), ((), ())),
                                    preferred_element_type=F32)
                    + lax.dot_general(qr, kr_b, (((1,), (1,)), ((), ())),
                                      preferred_element_type=F32)
                ) * SCALE
                m = jnp.max(scores, axis=-1, keepdims=True)
                e = jnp.exp(scores - m)
                pattn = (e / jnp.sum(e, axis=-1, keepdims=True)).astype(BF)
                o = jnp.dot(pattn, v, preferred_element_type=F32)
                obuf[my_y, pl.ds(b * S, S), pl.ds(h * Dh, Dh)] = o.astype(BF)

        for p in range(N_Y):
            @pl.when(my_y != p)
            def _():
                pltpu.make_async_remote_copy(
                    src_ref=obuf.at[my_y], dst_ref=obuf.at[my_y],
                    send_sem=o_s.at[p], recv_sem=o_r.at[my_y],
                    device_id=(my_x, p, my_z),
                    device_id_type=pl.DeviceIdType.MESH,
                ).start()

        out_ref[...] = jnp.dot(
            obuf[my_y], wo_ref[pl.ds(my_y * CW, CW), :].astype(BF),
            preferred_element_type=F32)

        for p in range(N_Y):
            @pl.when(my_y != p)
            def _():
                pltpu.make_async_remote_copy(
                    src_ref=obuf.at[p], dst_ref=obuf.at[p],
                    send_sem=o_s.at[p], recv_sem=o_r.at[p],
                    device_id=(my_x, p, my_z),
                    device_id_type=pl.DeviceIdType.MESH,
                ).wait_recv()
                out_ref[...] = out_ref[...] + jnp.dot(
                    obuf[p], wo_ref[pl.ds(p * CW, CW), :].astype(BF),
                    preferred_element_type=F32)

        for p in range(N_Y):
            @pl.when(my_y != p)
            def _():
                dev = (my_x, p, my_z)
                for sem, buf in ((c_s, cbuf), (wk_s, wkbuf),
                                 (wv_s, wvbuf), (o_s, obuf)):
                    pltpu.make_async_remote_copy(
                        src_ref=buf.at[my_y], dst_ref=buf.at[my_y],
                        send_sem=sem.at[p], recv_sem=o_r.at[p],
                        device_id=dev, device_id_type=pl.DeviceIdType.MESH,
                    ).wait_send()

    out = pl.pallas_call(
        body,
        out_shape=jax.ShapeDtypeStruct((BS, D), jnp.float32),
        in_specs=[pl.BlockSpec(memory_space=pltpu.VMEM)] * 8,
        out_specs=pl.BlockSpec(memory_space=pltpu.VMEM),
        scratch_shapes=[
            pltpu.VMEM((N_Y, BS, DC), BF),
            pltpu.VMEM((N_Y, DC, CW), jnp.float32),
            pltpu.VMEM((N_Y, DC, CW), jnp.float32),
            pltpu.VMEM((N_Y, BS, CW), BF),
            pltpu.SemaphoreType.DMA((N_Y,)),
            pltpu.SemaphoreType.DMA((N_Y,)),
            pltpu.SemaphoreType.DMA((N_Y,)),
            pltpu.SemaphoreType.DMA((N_Y,)),
            pltpu.SemaphoreType.DMA((N_Y,)),
            pltpu.SemaphoreType.DMA((N_Y,)),
            pltpu.SemaphoreType.DMA((N_Y,)),
            pltpu.SemaphoreType.DMA((N_Y,)),
        ],
        compiler_params=pltpu.CompilerParams(collective_id=0),
    )(x2, Wdkv, Wuk, Wuv, Wq, Wqr, Wkr, Wo)
    return out.reshape(B, S, D)


# device time: 40573 ns/iter; 5.2957x vs baseline; 1.1122x over previous
import jax
import jax.numpy as jnp
from jax import lax
from jax.experimental import pallas as pl
from jax.experimental.pallas import tpu as pltpu

N_Y = 4
B, S, H, Dh, Dr = 2, 256, 16, 64, 32
D = 1024
BS = B * S
HL = H // N_Y
CW = HL * Dh
RW = HL * Dr
DC = 64
SCALE = (Dh + Dr) ** -0.5
BF = jnp.bfloat16
F32 = jnp.float32


def kernel(x, Wdkv, Wuk, Wuv, Wq, Wqr, Wkr, Wo):
    x2 = x.reshape(BS, D)

    def body(x_ref, wdkv_ref, wuk_ref, wuv_ref, wq_ref, wqr_ref, wkr_ref,
             wo_ref, out_ref, cbuf, wksnd, wvsnd, wkbuf, wvbuf, obuf,
             c_s, c_r, wk_s, wk_r, wv_s, wv_r, o_s, o_r):
        my_x = lax.axis_index("x")
        my_y = lax.axis_index("y")
        my_z = lax.axis_index("z")

        barrier = pltpu.get_barrier_semaphore()
        for p in range(N_Y):
            @pl.when(my_y != p)
            def _():
                pl.semaphore_signal(
                    barrier, inc=1,
                    device_id=(my_x, p, my_z),
                    device_id_type=pl.DeviceIdType.MESH,
                )

        xv = x_ref[...].astype(BF)

        cbuf[my_y] = jnp.dot(xv, wdkv_ref[...].astype(BF),
                             preferred_element_type=F32).astype(BF)
        for p in range(N_Y):
            wksnd[p] = wuk_ref[:, pl.ds(p * CW, CW)].astype(BF)
            wvsnd[p] = wuv_ref[:, pl.ds(p * CW, CW)].astype(BF)
        wkbuf[my_y] = wksnd[my_y]
        wvbuf[my_y] = wvsnd[my_y]

        pl.semaphore_wait(barrier, N_Y - 1)

        for p in range(N_Y):
            @pl.when(my_y != p)
            def _():
                dev = (my_x, p, my_z)
                pltpu.make_async_remote_copy(
                    src_ref=cbuf.at[my_y], dst_ref=cbuf.at[my_y],
                    send_sem=c_s.at[p], recv_sem=c_r.at[my_y],
                    device_id=dev, device_id_type=pl.DeviceIdType.MESH,
                ).start()
                pltpu.make_async_remote_copy(
                    src_ref=wksnd.at[p], dst_ref=wkbuf.at[my_y],
                    send_sem=wk_s.at[p], recv_sem=wk_r.at[my_y],
                    device_id=dev, device_id_type=pl.DeviceIdType.MESH,
                ).start()
                pltpu.make_async_remote_copy(
                    src_ref=wvsnd.at[p], dst_ref=wvbuf.at[my_y],
                    send_sem=wv_s.at[p], recv_sem=wv_r.at[my_y],
                    device_id=dev, device_id_type=pl.DeviceIdType.MESH,
                ).start()

        q_y = jnp.dot(xv, wq_ref[:, pl.ds(my_y * CW, CW)].astype(BF),
                      preferred_element_type=F32).astype(BF)
        qr_y = jnp.dot(xv, wqr_ref[:, pl.ds(my_y * RW, RW)].astype(BF),
                       preferred_element_type=F32).astype(BF)
        kr_bf = jnp.dot(xv, wkr_ref[...].astype(BF),
                        preferred_element_type=F32).astype(BF)

        for p in range(N_Y):
            @pl.when(my_y != p)
            def _():
                dev = (my_x, p, my_z)
                pltpu.make_async_remote_copy(
                    src_ref=cbuf.at[p], dst_ref=cbuf.at[p],
                    send_sem=c_s.at[p], recv_sem=c_r.at[p],
                    device_id=dev, device_id_type=pl.DeviceIdType.MESH,
                ).wait_recv()
                pltpu.make_async_remote_copy(
                    src_ref=wkbuf.at[p], dst_ref=wkbuf.at[p],
                    send_sem=wk_s.at[p], recv_sem=wk_r.at[p],
                    device_id=dev, device_id_type=pl.DeviceIdType.MESH,
                ).wait_recv()
                pltpu.make_async_remote_copy(
                    src_ref=wvbuf.at[p], dst_ref=wvbuf.at[p],
                    send_sem=wv_s.at[p], recv_sem=wv_r.at[p],
                    device_id=dev, device_id_type=pl.DeviceIdType.MESH,
                ).wait_recv()

        c_full = jnp.concatenate([cbuf[p] for p in range(N_Y)], axis=1)
        wk_my = jnp.concatenate([wkbuf[p] for p in range(N_Y)], axis=0)
        wv_my = jnp.concatenate([wvbuf[p] for p in range(N_Y)], axis=0)
        k_y = jnp.dot(c_full, wk_my, preferred_element_type=F32).astype(BF)
        v_y = jnp.dot(c_full, wv_my, preferred_element_type=F32).astype(BF)

        wo_my = wo_ref[pl.ds(my_y * CW, CW), :].astype(BF)

        for b in range(B):
            kr_b = kr_bf[b * S:(b + 1) * S, :]
            for h in range(HL):
                q = q_y[b * S:(b + 1) * S, h * Dh:(h + 1) * Dh]
                qr = qr_y[b * S:(b + 1) * S, h * Dr:(h + 1) * Dr]
                k = k_y[b * S:(b + 1) * S, h * Dh:(h + 1) * Dh]
                v = v_y[b * S:(b + 1) * S, h * Dh:(h + 1) * Dh]
                scores = (
                    lax.dot_general(q, k, (((1,), (1,)), ((), ())),
                                    preferred_element_type=F32)
                    + lax.dot_general(qr, kr_b, (((1,), (1,)), ((), ())),
                                      preferred_element_type=F32)
                ) * SCALE
                m = jnp.max(scores, axis=-1, keepdims=True)
                e = jnp.exp(scores - m)
                pattn = (e / jnp.sum(e, axis=-1, keepdims=True)).astype(BF)
                o = jnp.dot(pattn, v, preferred_element_type=F32)
                obuf[my_y, pl.ds(b * S, S), pl.ds(h * Dh, Dh)] = o.astype(BF)

            for p in range(N_Y):
                @pl.when(my_y != p)
                def _():
                    pltpu.make_async_remote_copy(
                        src_ref=obuf.at[my_y, pl.ds(b * S, S)],
                        dst_ref=obuf.at[my_y, pl.ds(b * S, S)],
                        send_sem=o_s.at[b, p], recv_sem=o_r.at[b, my_y],
                        device_id=(my_x, p, my_z),
                        device_id_type=pl.DeviceIdType.MESH,
                    ).start()
            out_ref[pl.ds(b * S, S), :] = jnp.dot(
                obuf[my_y, pl.ds(b * S, S)], wo_my,
                preferred_element_type=F32)

        for p in range(N_Y):
            @pl.when(my_y != p)
            def _():
                for b in range(B):
                    pltpu.make_async_remote_copy(
                        src_ref=obuf.at[p, pl.ds(b * S, S)],
                        dst_ref=obuf.at[p, pl.ds(b * S, S)],
                        send_sem=o_s.at[b, p], recv_sem=o_r.at[b, p],
                        device_id=(my_x, p, my_z),
                        device_id_type=pl.DeviceIdType.MESH,
                    ).wait_recv()
                    out_ref[pl.ds(b * S, S), :] = (
                        out_ref[pl.ds(b * S, S), :]
                        + jnp.dot(obuf[p, pl.ds(b * S, S)],
                                  wo_ref[pl.ds(p * CW, CW), :].astype(BF),
                                  preferred_element_type=F32))

        for p in range(N_Y):
            @pl.when(my_y != p)
            def _():
                dev = (my_x, p, my_z)
                for sem, src in ((c_s.at[p], cbuf.at[my_y]),
                                 (wk_s.at[p], wksnd.at[p]),
                                 (wv_s.at[p], wvsnd.at[p]),
                                 (o_s.at[0, p], obuf.at[my_y, pl.ds(0, S)]),
                                 (o_s.at[1, p], obuf.at[my_y, pl.ds(S, S)])):
                    pltpu.make_async_remote_copy(
                        src_ref=src, dst_ref=src,
                        send_sem=sem, recv_sem=c_r.at[p],
                        device_id=dev, device_id_type=pl.DeviceIdType.MESH,
                    ).wait_send()

    out = pl.pallas_call(
        body,
        out_shape=jax.ShapeDtypeStruct((BS, D), jnp.float32),
        in_specs=[pl.BlockSpec(memory_space=pltpu.VMEM)] * 8,
        out_specs=pl.BlockSpec(memory_space=pltpu.VMEM),
        scratch_shapes=[
            pltpu.VMEM((N_Y, BS, DC), BF),
            pltpu.VMEM((N_Y, DC, CW), BF),
            pltpu.VMEM((N_Y, DC, CW), BF),
            pltpu.VMEM((N_Y, DC, CW), BF),
            pltpu.VMEM((N_Y, DC, CW), BF),
            pltpu.VMEM((N_Y, BS, CW), BF),
            pltpu.SemaphoreType.DMA((N_Y,)),
            pltpu.SemaphoreType.DMA((N_Y,)),
            pltpu.SemaphoreType.DMA((N_Y,)),
            pltpu.SemaphoreType.DMA((N_Y,)),
            pltpu.SemaphoreType.DMA((N_Y,)),
            pltpu.SemaphoreType.DMA((N_Y,)),
            pltpu.SemaphoreType.DMA((B, N_Y)),
            pltpu.SemaphoreType.DMA((B, N_Y)),
        ],
        compiler_params=pltpu.CompilerParams(collective_id=0),
    )(x2, Wdkv, Wuk, Wuv, Wq, Wqr, Wkr, Wo)
    return out.reshape(B, S, D)


# device time: 40541 ns/iter; 5.2998x vs baseline; 1.0008x over previous
import jax
import jax.numpy as jnp
from jax import lax
from jax.experimental import pallas as pl
from jax.experimental.pallas import tpu as pltpu

N_Y = 4
B, S, H, Dh, Dr = 2, 256, 16, 64, 32
D = 1024
BS = B * S
HL = H // N_Y
CW = HL * Dh
RW = HL * Dr
DC = 64
SCALE = (Dh + Dr) ** -0.5
BF = jnp.bfloat16
F32 = jnp.float32


def kernel(x, Wdkv, Wuk, Wuv, Wq, Wqr, Wkr, Wo):
    x2 = x.reshape(BS, D)

    def body(x_ref, wdkv_ref, wuk_ref, wuv_ref, wq_ref, wqr_ref, wkr_ref,
             wo_ref, out_ref, cbuf, wksnd, wvsnd, wkbuf, wvbuf, obuf,
             c_s, c_r, wk_s, wk_r, wv_s, wv_r, o_s, o_r):
        my_x = lax.axis_index("x")
        my_y = lax.axis_index("y")
        my_z = lax.axis_index("z")

        barrier = pltpu.get_barrier_semaphore()
        for p in range(N_Y):
            @pl.when(my_y != p)
            def _():
                pl.semaphore_signal(
                    barrier, inc=1,
                    device_id=(my_x, p, my_z),
                    device_id_type=pl.DeviceIdType.MESH,
                )

        xv = x_ref[...].astype(BF)

        cbuf[my_y] = jnp.dot(xv, wdkv_ref[...].astype(BF),
                             preferred_element_type=F32).astype(BF)
        for p in range(N_Y):
            wksnd[p] = wuk_ref[:, pl.ds(p * CW, CW)].astype(BF)
            wvsnd[p] = wuv_ref[:, pl.ds(p * CW, CW)].astype(BF)
        wkbuf[my_y] = wksnd[my_y]
        wvbuf[my_y] = wvsnd[my_y]

        pl.semaphore_wait(barrier, N_Y - 1)

        for p in range(N_Y):
            @pl.when(my_y != p)
            def _():
                dev = (my_x, p, my_z)
                pltpu.make_async_remote_copy(
                    src_ref=cbuf.at[my_y], dst_ref=cbuf.at[my_y],
                    send_sem=c_s.at[p], recv_sem=c_r.at[my_y],
                    device_id=dev, device_id_type=pl.DeviceIdType.MESH,
                ).start()
                pltpu.make_async_remote_copy(
                    src_ref=wksnd.at[p], dst_ref=wkbuf.at[my_y],
                    send_sem=wk_s.at[p], recv_sem=wk_r.at[my_y],
                    device_id=dev, device_id_type=pl.DeviceIdType.MESH,
                ).start()
                pltpu.make_async_remote_copy(
                    src_ref=wvsnd.at[p], dst_ref=wvbuf.at[my_y],
                    send_sem=wv_s.at[p], recv_sem=wv_r.at[my_y],
                    device_id=dev, device_id_type=pl.DeviceIdType.MESH,
                ).start()

        q_y = jnp.dot(xv, wq_ref[:, pl.ds(my_y * CW, CW)].astype(BF),
                      preferred_element_type=F32).astype(BF)
        qr_y = jnp.dot(xv, wqr_ref[:, pl.ds(my_y * RW, RW)].astype(BF),
                       preferred_element_type=F32).astype(BF)
        kr_bf = jnp.dot(xv, wkr_ref[...].astype(BF),
                        preferred_element_type=F32).astype(BF)

        for p in range(N_Y):
            @pl.when(my_y != p)
            def _():
                dev = (my_x, p, my_z)
                pltpu.make_async_remote_copy(
                    src_ref=cbuf.at[p], dst_ref=cbuf.at[p],
                    send_sem=c_s.at[p], recv_sem=c_r.at[p],
                    device_id=dev, device_id_type=pl.DeviceIdType.MESH,
                ).wait_recv()
                pltpu.make_async_remote_copy(
                    src_ref=wkbuf.at[p], dst_ref=wkbuf.at[p],
                    send_sem=wk_s.at[p], recv_sem=wk_r.at[p],
                    device_id=dev, device_id_type=pl.DeviceIdType.MESH,
                ).wait_recv()
                pltpu.make_async_remote_copy(
                    src_ref=wvbuf.at[p], dst_ref=wvbuf.at[p],
                    send_sem=wv_s.at[p], recv_sem=wv_r.at[p],
                    device_id=dev, device_id_type=pl.DeviceIdType.MESH,
                ).wait_recv()

        wo_bf = wo_ref[...].astype(BF)

        c_full = jnp.concatenate([cbuf[p] for p in range(N_Y)], axis=1)
        wk_my = jnp.concatenate([wkbuf[p] for p in range(N_Y)], axis=0)
        wv_my = jnp.concatenate([wvbuf[p] for p in range(N_Y)], axis=0)
        k_y = jnp.dot(c_full, wk_my, preferred_element_type=F32).astype(BF)
        v_y = jnp.dot(c_full, wv_my, preferred_element_type=F32).astype(BF)

        for b in range(B):
            kr_b = kr_bf[b * S:(b + 1) * S, :]
            for h in range(HL):
                q = q_y[b * S:(b + 1) * S, h * Dh:(h + 1) * Dh]
                qr = qr_y[b * S:(b + 1) * S, h * Dr:(h + 1) * Dr]
                k = k_y[b * S:(b + 1) * S, h * Dh:(h + 1) * Dh]
                v = v_y[b * S:(b + 1) * S, h * Dh:(h + 1) * Dh]
                qcat = jnp.concatenate([q, qr], axis=1)
                kcat = jnp.concatenate([k, kr_b], axis=1)
                scores = lax.dot_general(
                    qcat, kcat, (((1,), (1,)), ((), ())),
                    preferred_element_type=F32) * SCALE
                m = jnp.max(scores, axis=-1, keepdims=True)
                e = jnp.exp(scores - m)
                pattn = (e / jnp.sum(e, axis=-1, keepdims=True)).astype(BF)
                o = jnp.dot(pattn, v, preferred_element_type=F32)
                obuf[my_y, pl.ds(b * S, S), pl.ds(h * Dh, Dh)] = o.astype(BF)

            for p in range(N_Y):
                @pl.when(my_y != p)
                def _():
                    pltpu.make_async_remote_copy(
                        src_ref=obuf.at[my_y, pl.ds(b * S, S)],
                        dst_ref=obuf.at[my_y, pl.ds(b * S, S)],
                        send_sem=o_s.at[b, p], recv_sem=o_r.at[b, my_y],
                        device_id=(my_x, p, my_z),
                        device_id_type=pl.DeviceIdType.MESH,
                    ).start()

        for b in range(B):
            for p in range(N_Y):
                @pl.when(my_y != p)
                def _():
                    pltpu.make_async_remote_copy(
                        src_ref=obuf.at[p, pl.ds(b * S, S)],
                        dst_ref=obuf.at[p, pl.ds(b * S, S)],
                        send_sem=o_s.at[b, p], recv_sem=o_r.at[b, p],
                        device_id=(my_x, p, my_z),
                        device_id_type=pl.DeviceIdType.MESH,
                    ).wait_recv()
            out_ref[pl.ds(b * S, S), :] = sum(
                jnp.dot(obuf[p, pl.ds(b * S, S)],
                        wo_bf[p * CW:(p + 1) * CW, :],
                        preferred_element_type=F32)
                for p in range(N_Y))

        for p in range(N_Y):
            @pl.when(my_y != p)
            def _():
                dev = (my_x, p, my_z)
                for sem, src in ((c_s.at[p], cbuf.at[my_y]),
                                 (wk_s.at[p], wksnd.at[p]),
                                 (wv_s.at[p], wvsnd.at[p]),
                                 (o_s.at[0, p], obuf.at[my_y, pl.ds(0, S)]),
                                 (o_s.at[1, p], obuf.at[my_y, pl.ds(S, S)])):
                    pltpu.make_async_remote_copy(
                        src_ref=src, dst_ref=src,
                        send_sem=sem, recv_sem=c_r.at[p],
                        device_id=dev, device_id_type=pl.DeviceIdType.MESH,
                    ).wait_send()

    out = pl.pallas_call(
        body,
        out_shape=jax.ShapeDtypeStruct((BS, D), jnp.float32),
        in_specs=[pl.BlockSpec(memory_space=pltpu.VMEM)] * 8,
        out_specs=pl.BlockSpec(memory_space=pltpu.VMEM),
        scratch_shapes=[
            pltpu.VMEM((N_Y, BS, DC), BF),
            pltpu.VMEM((N_Y, DC, CW), BF),
            pltpu.VMEM((N_Y, DC, CW), BF),
            pltpu.VMEM((N_Y, DC, CW), BF),
            pltpu.VMEM((N_Y, DC, CW), BF),
            pltpu.VMEM((N_Y, BS, CW), BF),
            pltpu.SemaphoreType.DMA((N_Y,)),
            pltpu.SemaphoreType.DMA((N_Y,)),
            pltpu.SemaphoreType.DMA((N_Y,)),
            pltpu.SemaphoreType.DMA((N_Y,)),
            pltpu.SemaphoreType.DMA((N_Y,)),
            pltpu.SemaphoreType.DMA((N_Y,)),
            pltpu.SemaphoreType.DMA((B, N_Y)),
            pltpu.SemaphoreType.DMA((B, N_Y)),
        ],
        compiler_params=pltpu.CompilerParams(collective_id=0),
    )(x2, Wdkv, Wuk, Wuv, Wq, Wqr, Wkr, Wo)
    return out.reshape(B, S, D)
